# baseline (speedup 1.0000x reference)
import functools

import jax
import jax.numpy as jnp
from jax import lax
from jax.experimental import pallas as pl
from jax.experimental.pallas import tpu as pltpu

T, D, O, L, R = 8192, 2048, 2048, 16, 16
TT = 512


def _fused_kernel(x_ref, wt_ref, a_ref, b_ref, bse_ref, idx_ref, out_ref):
    idx = idx_ref[0, 0, :]

    KC = 512
    s = None
    y = None
    for k in range(D // KC):
        sl = slice(k * KC, (k + 1) * KC)
        xk = x_ref[:, sl].astype(jnp.bfloat16)
        ak = a_ref[:, sl].astype(jnp.bfloat16)
        sk = lax.dot_general(xk, ak, (((1,), (1,)), ((), ())),
                             preferred_element_type=jnp.float32)
        yk = lax.dot_general(xk, wt_ref[sl, :], (((1,), (0,)), ((), ())),
                             preferred_element_type=jnp.float32)
        s = sk if s is None else s + sk
        y = yk if y is None else y + yk

    colgrp = lax.broadcasted_iota(jnp.int32, (TT, L * R), 1) // R
    z = jnp.where(colgrp == idx[:, None], s, 0.0).astype(jnp.bfloat16)

    b = b_ref[:].astype(jnp.bfloat16)
    y = y + lax.dot_general(z, b, (((1,), (0,)), ((), ())),
                            preferred_element_type=jnp.float32)

    lane = lax.broadcasted_iota(jnp.int32, (TT, L + 1), 1)
    sel = jnp.where(idx < 0, L, idx)
    onehot = (lane == sel[:, None]).astype(jnp.bfloat16)
    y = y + lax.dot_general(onehot, bse_ref[:], (((1,), (0,)), ((), ())),
                            preferred_element_type=jnp.float32)

    out_ref[:] = y


@jax.jit
def kernel(input_, W, bias, lora_a_stacked, lora_b_stacked, bias_stacked,
           token_lora_indices):
    wt = W.T.astype(jnp.bfloat16)
    a_cat = lora_a_stacked.reshape(L * R, D)
    b_cat = jnp.transpose(lora_b_stacked, (0, 2, 1)).reshape(L * R, O)
    bse = jnp.concatenate([bias_stacked + bias[None, :], bias[None, :]],
                          axis=0).astype(jnp.bfloat16)
    idx3d = token_lora_indices.astype(jnp.int32).reshape(T // TT, 1, TT)

    grid = (T // TT,)
    return pl.pallas_call(
        _fused_kernel,
        grid=grid,
        in_specs=[
            pl.BlockSpec((TT, D), lambda i: (i, 0)),
            pl.BlockSpec((D, O), lambda i: (0, 0)),
            pl.BlockSpec((L * R, D), lambda i: (0, 0)),
            pl.BlockSpec((L * R, O), lambda i: (0, 0)),
            pl.BlockSpec((L + 1, O), lambda i: (0, 0)),
            pl.BlockSpec((1, 1, TT), lambda i: (i, 0, 0)),
        ],
        out_specs=pl.BlockSpec((TT, O), lambda i: (i, 0)),
        out_shape=jax.ShapeDtypeStruct((T, O), jnp.float32),
        compiler_params=pltpu.CompilerParams(
            dimension_semantics=("parallel",)),
    )(input_, wt, a_cat, b_cat, bse, idx3d)


# speedup vs baseline: 1.0897x; 1.0897x over previous
import functools

import jax
import jax.numpy as jnp
from jax import lax
from jax.experimental import pallas as pl
from jax.experimental.pallas import tpu as pltpu

T, D, O, L, R = 8192, 2048, 2048, 16, 16
TT = 512


def _fused_kernel(x_ref, w_ref, a_ref, b_ref, bse_ref, idx_ref, out_ref):
    idx = idx_ref[0, 0, :]

    KC = 512
    s = None
    y = None
    for k in range(D // KC):
        sl = slice(k * KC, (k + 1) * KC)
        xk = x_ref[:, sl].astype(jnp.bfloat16)
        ak = a_ref[:, sl].astype(jnp.bfloat16)
        wk = w_ref[:, sl].astype(jnp.bfloat16)
        sk = lax.dot_general(xk, ak, (((1,), (1,)), ((), ())),
                             preferred_element_type=jnp.float32)
        yk = lax.dot_general(xk, wk, (((1,), (1,)), ((), ())),
                             preferred_element_type=jnp.float32)
        s = sk if s is None else s + sk
        y = yk if y is None else y + yk

    colgrp = lax.broadcasted_iota(jnp.int32, (TT, L * R), 1) // R
    z = jnp.where(colgrp == idx[:, None], s, 0.0).astype(jnp.bfloat16)

    b = b_ref[:].astype(jnp.bfloat16)
    y = y + lax.dot_general(z, b, (((1,), (0,)), ((), ())),
                            preferred_element_type=jnp.float32)

    lane = lax.broadcasted_iota(jnp.int32, (TT, L + 1), 1)
    sel = jnp.where(idx < 0, L, idx)
    onehot = (lane == sel[:, None]).astype(jnp.bfloat16)
    y = y + lax.dot_general(onehot, bse_ref[:], (((1,), (0,)), ((), ())),
                            preferred_element_type=jnp.float32)

    out_ref[:] = y


@jax.jit
def kernel(input_, W, bias, lora_a_stacked, lora_b_stacked, bias_stacked,
           token_lora_indices):
    a_cat = lora_a_stacked.reshape(L * R, D)
    b_cat = jnp.transpose(lora_b_stacked, (0, 2, 1)).reshape(L * R, O)
    bse = jnp.concatenate([bias_stacked + bias[None, :], bias[None, :]],
                          axis=0).astype(jnp.bfloat16)
    idx3d = token_lora_indices.astype(jnp.int32).reshape(T // TT, 1, TT)

    grid = (T // TT,)
    return pl.pallas_call(
        _fused_kernel,
        grid=grid,
        in_specs=[
            pl.BlockSpec((TT, D), lambda i: (i, 0)),
            pl.BlockSpec((O, D), lambda i: (0, 0)),
            pl.BlockSpec((L * R, D), lambda i: (0, 0)),
            pl.BlockSpec((L * R, O), lambda i: (0, 0)),
            pl.BlockSpec((L + 1, O), lambda i: (0, 0)),
            pl.BlockSpec((1, 1, TT), lambda i: (i, 0, 0)),
        ],
        out_specs=pl.BlockSpec((TT, O), lambda i: (i, 0)),
        out_shape=jax.ShapeDtypeStruct((T, O), jnp.float32),
        compiler_params=pltpu.CompilerParams(
            dimension_semantics=("parallel",)),
    )(input_, W, a_cat, b_cat, bse, idx3d)


# speedup vs baseline: 1.0935x; 1.0036x over previous
import functools

import jax
import jax.numpy as jnp
from jax import lax
from jax.experimental import pallas as pl
from jax.experimental.pallas import tpu as pltpu

T, D, O, L, R = 8192, 2048, 2048, 16, 16
TT = 512


def _fused_kernel(x_ref, w_ref, a_ref, b_ref, bse_ref, idx_ref, out_ref):
    idx = idx_ref[0, 0, :]

    KC = 1024
    s = None
    y = None
    for k in range(D // KC):
        sl = slice(k * KC, (k + 1) * KC)
        xk = x_ref[:, sl].astype(jnp.bfloat16)
        ak = a_ref[:, sl].astype(jnp.bfloat16)
        wk = w_ref[:, sl].astype(jnp.bfloat16)
        sk = lax.dot_general(xk, ak, (((1,), (1,)), ((), ())),
                             preferred_element_type=jnp.float32)
        yk = lax.dot_general(xk, wk, (((1,), (1,)), ((), ())),
                             preferred_element_type=jnp.float32)
        s = sk if s is None else s + sk
        y = yk if y is None else y + yk

    colgrp = lax.broadcasted_iota(jnp.int32, (TT, L * R), 1) // R
    z = jnp.where(colgrp == idx[:, None], s, 0.0).astype(jnp.bfloat16)

    b = b_ref[:].astype(jnp.bfloat16)
    y = y + lax.dot_general(z, b, (((1,), (0,)), ((), ())),
                            preferred_element_type=jnp.float32)

    lane = lax.broadcasted_iota(jnp.int32, (TT, L + 1), 1)
    sel = jnp.where(idx < 0, L, idx)
    onehot = (lane == sel[:, None]).astype(jnp.bfloat16)
    y = y + lax.dot_general(onehot, bse_ref[:], (((1,), (0,)), ((), ())),
                            preferred_element_type=jnp.float32)

    out_ref[:] = y


@jax.jit
def kernel(input_, W, bias, lora_a_stacked, lora_b_stacked, bias_stacked,
           token_lora_indices):
    a_cat = lora_a_stacked.reshape(L * R, D)
    b_cat = jnp.transpose(lora_b_stacked, (0, 2, 1)).reshape(L * R, O)
    bse = jnp.concatenate([bias_stacked + bias[None, :], bias[None, :]],
                          axis=0).astype(jnp.bfloat16)
    idx3d = token_lora_indices.astype(jnp.int32).reshape(T // TT, 1, TT)

    grid = (T // TT,)
    return pl.pallas_call(
        _fused_kernel,
        grid=grid,
        in_specs=[
            pl.BlockSpec((TT, D), lambda i: (i, 0)),
            pl.BlockSpec((O, D), lambda i: (0, 0)),
            pl.BlockSpec((L * R, D), lambda i: (0, 0)),
            pl.BlockSpec((L * R, O), lambda i: (0, 0)),
            pl.BlockSpec((L + 1, O), lambda i: (0, 0)),
            pl.BlockSpec((1, 1, TT), lambda i: (i, 0, 0)),
        ],
        out_specs=pl.BlockSpec((TT, O), lambda i: (i, 0)),
        out_shape=jax.ShapeDtypeStruct((T, O), jnp.float32),
        compiler_params=pltpu.CompilerParams(
            dimension_semantics=("parallel",)),
    )(input_, W, a_cat, b_cat, bse, idx3d)


# speedup vs baseline: 1.0991x; 1.0050x over previous
import functools

import jax
import jax.numpy as jnp
from jax import lax
from jax.experimental import pallas as pl
from jax.experimental.pallas import tpu as pltpu

T, D, O, L, R = 8192, 2048, 2048, 16, 16
TT = 512


def _fused_kernel(x_ref, w_ref, bias_ref, a_ref, b_ref, bs_ref, idx_ref,
                  out_ref):
    idx = idx_ref[0, 0, :]

    KC = 512
    s = None
    y = None
    for k in range(D // KC):
        sl = slice(k * KC, (k + 1) * KC)
        xk = x_ref[:, sl].astype(jnp.bfloat16)
        ak = a_ref[:, sl].astype(jnp.bfloat16)
        wk = w_ref[:, sl].astype(jnp.bfloat16)
        sk = lax.dot_general(xk, ak, (((1,), (1,)), ((), ())),
                             preferred_element_type=jnp.float32)
        yk = lax.dot_general(xk, wk, (((1,), (1,)), ((), ())),
                             preferred_element_type=jnp.float32)
        s = sk if s is None else s + sk
        y = yk if y is None else y + yk

    colgrp = lax.broadcasted_iota(jnp.int32, (TT, L * R), 1) // R
    z = jnp.where(colgrp == idx[:, None], s, 0.0).astype(jnp.bfloat16)

    b = b_ref[:].astype(jnp.bfloat16)
    y = y + lax.dot_general(z, b, (((1,), (0,)), ((), ())),
                            preferred_element_type=jnp.float32)

    lane = lax.broadcasted_iota(jnp.int32, (TT, L), 1)
    onehot = (lane == idx[:, None]).astype(jnp.bfloat16)
    y = y + lax.dot_general(onehot, bs_ref[:].astype(jnp.bfloat16),
                            (((1,), (0,)), ((), ())),
                            preferred_element_type=jnp.float32)

    out_ref[:] = y + bias_ref[:]


@jax.jit
def kernel(input_, W, bias, lora_a_stacked, lora_b_stacked, bias_stacked,
           token_lora_indices):
    a_cat = lora_a_stacked.reshape(L * R, D)
    b_cat = jnp.transpose(lora_b_stacked, (0, 2, 1)).reshape(L * R, O)
    bias2d = bias.reshape(1, O)
    idx3d = token_lora_indices.astype(jnp.int32).reshape(T // TT, 1, TT)

    grid = (T // TT,)
    return pl.pallas_call(
        _fused_kernel,
        grid=grid,
        in_specs=[
            pl.BlockSpec((TT, D), lambda i: (i, 0)),
            pl.BlockSpec((O, D), lambda i: (0, 0)),
            pl.BlockSpec((1, O), lambda i: (0, 0)),
            pl.BlockSpec((L * R, D), lambda i: (0, 0)),
            pl.BlockSpec((L * R, O), lambda i: (0, 0)),
            pl.BlockSpec((L, O), lambda i: (0, 0)),
            pl.BlockSpec((1, 1, TT), lambda i: (i, 0, 0)),
        ],
        out_specs=pl.BlockSpec((TT, O), lambda i: (i, 0)),
        out_shape=jax.ShapeDtypeStruct((T, O), jnp.float32),
        compiler_params=pltpu.CompilerParams(
            dimension_semantics=("parallel",)),
    )(input_, W, bias2d, a_cat, b_cat, bias_stacked, idx3d)
